# baseline (speedup 1.0000x reference)
import functools

import jax
import jax.numpy as jnp
from jax import lax
from jax.experimental import pallas as pl
from jax.experimental.pallas import tpu as pltpu
from jax.experimental.pallas import tpu_sc as plsc

B = 16384
F = 64
L = 16
NC = 2
NS = 16
NW = NC * NS
BPW = B // NW
CHUNK = 128
NCHUNK = BPW // CHUNK
GROUPS = BPW // L


@functools.partial(
    pl.kernel,
    mesh=plsc.VectorSubcoreMesh(core_axis_name="c", subcore_axis_name="s"),
    out_type=jax.ShapeDtypeStruct((B,), jnp.float32),
    compiler_params=pltpu.CompilerParams(needs_layout_passes=False,
                                         use_tc_tiling_on_sc=False),
    scratch_types=[
        pltpu.VMEM((NCHUNK, CHUNK), jnp.int32),
        pltpu.VMEM((NCHUNK, CHUNK), jnp.int32),
        pltpu.VMEM((BPW, F), jnp.float32),
        pltpu.VMEM((BPW, F), jnp.float32),
        pltpu.VMEM((BPW,), jnp.float32),
        pltpu.VMEM((L * L,), jnp.float32),
        pltpu.SemaphoreType.DMA,
        pltpu.SemaphoreType.DMA,
    ],
)
def _usmf(users_hbm, items_hbm, ue_hbm, ie_hbm, out_hbm,
          uidx_v, iidx_v, urows_v, irows_v, out_v, stage_v, sem_u, sem_i):
    wid = lax.axis_index("s") * NC + lax.axis_index("c")
    base = wid * BPW

    for j in range(NCHUNK):
        pltpu.sync_copy(users_hbm.at[pl.ds(base + j * CHUNK, CHUNK)],
                        uidx_v.at[j])
        pltpu.sync_copy(items_hbm.at[pl.ds(base + j * CHUNK, CHUNK)],
                        iidx_v.at[j])

    copies = []
    for j in range(NCHUNK):
        copies.append(pltpu.async_copy(
            ue_hbm.at[uidx_v.at[j]],
            urows_v.at[pl.ds(j * CHUNK, CHUNK)], sem_u))
        copies.append(pltpu.async_copy(
            ie_hbm.at[iidx_v.at[j]],
            irows_v.at[pl.ds(j * CHUNK, CHUNK)], sem_i))
    for c in copies:
        c.wait()

    lane = lax.iota(jnp.int32, L)
    colbase = lane * L

    def group_body(g, carry):
        row0 = g * L
        for r in range(L):
            row = row0 + r
            s = jnp.zeros((L,), jnp.float32)
            for q in range(F // L):
                u = urows_v[row, pl.ds(q * L, L)]
                v = irows_v[row, pl.ds(q * L, L)]
                s = s + u * v
            stage_v[pl.ds(r * L, L)] = s
        acc = jnp.zeros((L,), jnp.float32)
        for k in range(L):
            acc = acc + plsc.load_gather(stage_v, [colbase + k])
        out_v[pl.ds(row0, L)] = acc
        return carry

    lax.fori_loop(0, GROUPS, group_body, 0)

    pltpu.sync_copy(out_v, out_hbm.at[pl.ds(base, BPW)])


def kernel(users, items, user_embeddings, item_embeddings):
    users = users.astype(jnp.int32)
    items = items.astype(jnp.int32)
    return _usmf(users, items, user_embeddings, item_embeddings)


# speedup vs baseline: 2.1365x; 2.1365x over previous
import functools

import jax
import jax.numpy as jnp
from jax import lax
from jax.experimental import pallas as pl
from jax.experimental.pallas import tpu as pltpu
from jax.experimental.pallas import tpu_sc as plsc

B = 16384
F = 64
L = 16
NC = 2
NS = 16
NW = NC * NS
BPW = B // NW
NTC = 7813
TPC = 245
SLABC = 4
SLABW = SLABC * 128
CHROWS = 128
HITCAP = 1040
SENT = B
ROWS_OUT = B + CHROWS
GROUPS = BPW // L

_MESH = plsc.VectorSubcoreMesh(core_axis_name="c", subcore_axis_name="s")
_PARAMS = pltpu.CompilerParams(needs_layout_passes=False,
                               use_tc_tiling_on_sc=True)


@functools.partial(
    pl.kernel,
    mesh=_MESH,
    out_type=jax.ShapeDtypeStruct((ROWS_OUT, 2 * F), jnp.float32),
    compiler_params=_PARAMS,
    scratch_types=[
        pltpu.VMEM((B,), jnp.int32),
        pltpu.VMEM((HITCAP,), jnp.int32),
        pltpu.VMEM((F, SLABW), jnp.float32),
        pltpu.VMEM((2 * CHROWS, 2 * F), jnp.float32),
        pltpu.VMEM((2, CHROWS), jnp.int32),
        pltpu.SMEM((8,), jnp.int32),
        pltpu.SemaphoreType.DMA,
        pltpu.SemaphoreType.DMA,
    ],
)
def _gather_cols(idx_hbm, tabt_hbm, rows_hbm,
                 idx_v, hits_v, slab_v, stage_v, tags_v, st_s,
                 sem_slab, sem_scat):
    wid = lax.axis_index("s") * NC + lax.axis_index("c")
    c_lo = wid * TPC
    u_lo = c_lo * 128
    ncols = jnp.minimum(TPC, NTC - c_lo)
    nslab = (ncols + SLABC - 1) // SLABC
    lanes = lax.iota(jnp.int32, L)
    fvecs = [lanes + q * L for q in range(F // L)]

    st_s[0] = 0
    st_s[1] = 0
    st_s[2] = 0

    pltpu.sync_copy(idx_hbm, idx_v)

    def coll(k, cnt):
        v = idx_v[pl.ds(k * L, L)]
        urel = v - u_lo
        m = jnp.logical_and(urel >= 0, urel < ncols * 128)
        pack = jnp.bitwise_or(jnp.left_shift(k * L + lanes, 15),
                              jnp.bitwise_and(urel, 32767))
        plsc.store_compressed(hits_v.at[pl.ds(cnt, L)], pack, mask=m)
        return cnt + plsc.all_reduce_population_count(m)[0]

    cnt = lax.fori_loop(0, B // L, coll, 0)

    npad = (CHROWS - jnp.bitwise_and(cnt, CHROWS - 1)) & (CHROWS - 1)

    def pad(i, carry):
        ph = jnp.bitwise_or(jnp.int32(1 << 30), i * L + lanes)
        cur = jnp.where(i * L + lanes < npad, ph, -1)
        hits_v[pl.ds(cnt + i * L, L)] = cur
        return carry

    lax.fori_loop(0, CHROWS // L, pad, 0)
    total = cnt + npad
    hits_v[pl.ds(total, L)] = jnp.full((L,), -1, jnp.int32)
    nvec = (total + L - 1) // L

    def slab_body(s, carry):
        c_eff = jnp.minimum(c_lo + s * SLABC, NTC - SLABC)
        pltpu.sync_copy(tabt_hbm.at[:, pl.ds(c_eff * 128, SLABW)], slab_v)
        reloff = (c_eff - c_lo) * 128

        def scan(j, carry2):
            hv = hits_v[pl.ds(j * L, L)]
            urel16 = jnp.bitwise_and(hv, 32767)
            is_ph = hv >= (1 << 30)
            m0 = jnp.logical_and(
                jnp.logical_or(jnp.right_shift(urel16, 9) == s,
                               jnp.logical_and(is_ph, s == 0)),
                hv >= 0)

            pc = plsc.all_reduce_population_count(m0)[0]

            @pl.when(pc > 0)
            def _():
                def hit_loop(h, m):
                    lane = plsc.all_reduce_ffs(m)[0]
                    pack = hv.at[jnp.broadcast_to(lane, (L,))].get(
                        mode="promise_in_bounds")[0]
                    phantom = pack >= (1 << 30)
                    b = jnp.where(phantom,
                                  SENT + jnp.bitwise_and(pack, CHROWS - 1),
                                  jnp.right_shift(pack, 15))
                    ul = jnp.where(phantom, 0,
                                   jnp.bitwise_and(pack, 32767) - reloff)
                    rs = st_s[0]
                    slot = jnp.bitwise_and(rs, 2 * CHROWS - 1)
                    p = jnp.right_shift(jnp.bitwise_and(rs, 2 * CHROWS - 1),
                                        7)
                    @pl.when(jnp.logical_and(
                        jnp.bitwise_and(rs, CHROWS - 1) == 0,
                        rs >= 2 * CHROWS))
                    def _d():
                        pltpu.make_async_copy(
                            rows_hbm.at[pl.ds(0, CHROWS)],
                            stage_v.at[pl.ds(p * CHROWS, CHROWS)],
                            sem_scat).wait()
                        st_s[2] = st_s[2] + 1

                    ulv = jnp.broadcast_to(ul, (L,))
                    for q in range(F // L):
                        colq = plsc.load_gather(slab_v, [fvecs[q], ulv])
                        stage_v[slot, pl.ds(q * L, L)] = colq
                    wnd = jnp.left_shift(
                        jnp.right_shift(jnp.bitwise_and(rs, CHROWS - 1), 4),
                        4)
                    tw = tags_v[p, pl.ds(wnd, L)]
                    tags_v[p, pl.ds(wnd, L)] = jnp.where(
                        lanes == jnp.bitwise_and(rs, L - 1), b, tw)
                    st_s[0] = rs + 1

                    @pl.when(jnp.bitwise_and(rs, CHROWS - 1) == CHROWS - 1)
                    def _f():
                        pltpu.async_copy(
                            stage_v.at[pl.ds(p * CHROWS, CHROWS)],
                            rows_hbm.at[tags_v.at[p]],
                            sem_scat)
                        st_s[1] = st_s[1] + 1

                    return jnp.logical_and(m, lanes != lane)

                lax.fori_loop(0, pc, hit_loop, m0)

            return carry2

        lax.fori_loop(0, nvec, scan, 0)
        return carry

    lax.fori_loop(0, nslab, slab_body, 0)

    def drain(i, carry):
        pltpu.make_async_copy(
            rows_hbm.at[pl.ds(0, CHROWS)],
            stage_v.at[pl.ds(0, CHROWS)],
            sem_scat).wait()
        return carry

    lax.fori_loop(0, st_s[1] - st_s[2], drain, 0)


@functools.partial(
    pl.kernel,
    mesh=_MESH,
    out_type=jax.ShapeDtypeStruct((B,), jnp.float32),
    compiler_params=_PARAMS,
    scratch_types=[
        pltpu.VMEM((CHROWS, 2 * F), jnp.float32),
        pltpu.VMEM((CHROWS, 2 * F), jnp.float32),
        pltpu.VMEM((BPW,), jnp.float32),
        pltpu.VMEM((L * L,), jnp.float32),
    ],
)
def _dot_rows(urows_hbm, irows_hbm, out_hbm,
              uchunk_v, ichunk_v, out_v, stage_v):
    wid = lax.axis_index("s") * NC + lax.axis_index("c")
    base = wid * BPW
    lane = lax.iota(jnp.int32, L)
    colbase = lane * L

    def chunk_body(j, carry):
        c0 = j * CHROWS
        pltpu.sync_copy(urows_hbm.at[pl.ds(base + c0, CHROWS)], uchunk_v)
        pltpu.sync_copy(irows_hbm.at[pl.ds(base + c0, CHROWS)], ichunk_v)

        def group_body(g, carry2):
            for r in range(L):
                row = g * L + r
                s = jnp.zeros((L,), jnp.float32)
                for q in range(F // L):
                    u = uchunk_v[row, pl.ds(q * L, L)]
                    v = ichunk_v[row, pl.ds(q * L, L)]
                    s = s + u * v
                stage_v[pl.ds(r * L, L)] = s
            acc = jnp.zeros((L,), jnp.float32)
            for k in range(L):
                acc = acc + plsc.load_gather(stage_v, [colbase + k])
            out_v[pl.ds(c0 + g * L, L)] = acc
            return carry2

        lax.fori_loop(0, CHROWS // L, group_body, 0)
        return carry

    lax.fori_loop(0, BPW // CHROWS, chunk_body, 0)

    pltpu.sync_copy(out_v, out_hbm.at[pl.ds(base, BPW)])


def kernel(users, items, user_embeddings, item_embeddings):
    users = users.astype(jnp.int32)
    items = items.astype(jnp.int32)
    urows = _gather_cols(users, user_embeddings.T)
    irows = _gather_cols(items, item_embeddings.T)
    return _dot_rows(urows, irows)


# speedup vs baseline: 3.4598x; 1.6194x over previous
import functools

import jax
import jax.numpy as jnp
from jax import lax
from jax.experimental import pallas as pl
from jax.experimental.pallas import tpu as pltpu
from jax.experimental.pallas import tpu_sc as plsc

B = 16384
F = 64
L = 16
NC = 2
NS = 16
NW = NC * NS
BPW = B // NW
NTC = 7813
TPC = 245
SLABC = 4
SLABW = SLABC * 128
CHROWS = 128
HITCAP = 1040
SENT = B
ROWS_OUT = B + CHROWS
GROUPS = BPW // L

_MESH = plsc.VectorSubcoreMesh(core_axis_name="c", subcore_axis_name="s")
_PARAMS = pltpu.CompilerParams(needs_layout_passes=False,
                               use_tc_tiling_on_sc=True)


@functools.partial(
    pl.kernel,
    mesh=_MESH,
    out_type=jax.ShapeDtypeStruct((ROWS_OUT, 2 * F), jnp.float32),
    compiler_params=_PARAMS,
    scratch_types=[
        pltpu.VMEM((B,), jnp.int32),
        pltpu.VMEM((HITCAP,), jnp.int32),
        pltpu.VMEM((2, F, SLABW), jnp.float32),
        pltpu.VMEM((2 * CHROWS, 2 * F), jnp.float32),
        pltpu.VMEM((2, CHROWS), jnp.int32),
        pltpu.SMEM((8,), jnp.int32),
        pltpu.SemaphoreType.DMA,
        pltpu.SemaphoreType.DMA,
        pltpu.SemaphoreType.DMA,
    ],
)
def _gather_cols(idx_hbm, tabt_hbm, rows_hbm,
                 idx_v, hits_v, slab_v, stage_v, tags_v, st_s,
                 sem_sl0, sem_sl1, sem_scat):
    wid = lax.axis_index("s") * NC + lax.axis_index("c")
    c_lo = wid * TPC
    u_lo = c_lo * 128
    ncols = jnp.minimum(TPC, NTC - c_lo)
    nslab = (ncols + SLABC - 1) // SLABC
    lanes = lax.iota(jnp.int32, L)
    fvecs = [lanes + q * L for q in range(F // L)]

    st_s[0] = 0
    st_s[1] = 0
    st_s[2] = 0

    pltpu.sync_copy(idx_hbm, idx_v)

    def coll(k, cnt):
        v = idx_v[pl.ds(k * L, L)]
        urel = v - u_lo
        m = jnp.logical_and(urel >= 0, urel < ncols * 128)
        pack = jnp.bitwise_or(jnp.left_shift(k * L + lanes, 15),
                              jnp.bitwise_and(urel, 32767))
        plsc.store_compressed(hits_v.at[pl.ds(cnt, L)], pack, mask=m)
        return cnt + plsc.all_reduce_population_count(m)[0]

    cnt = lax.fori_loop(0, B // L, coll, 0)

    npad = (CHROWS - jnp.bitwise_and(cnt, CHROWS - 1)) & (CHROWS - 1)

    def pad(i, carry):
        ph = jnp.bitwise_or(jnp.int32(1 << 30), i * L + lanes)
        cur = jnp.where(i * L + lanes < npad, ph, -1)
        hits_v[pl.ds(cnt + i * L, L)] = cur
        return carry

    lax.fori_loop(0, CHROWS // L, pad, 0)
    total = cnt + npad
    hits_v[pl.ds(total, L)] = jnp.full((L,), -1, jnp.int32)
    nvec = (total + L - 1) // L

    def fire(s_next, pb_static):
        c_eff = jnp.minimum(c_lo + s_next * SLABC, NTC - SLABC)
        pltpu.async_copy(tabt_hbm.at[:, pl.ds(c_eff * 128, SLABW)],
                         slab_v.at[pb_static],
                         sem_sl0 if pb_static == 0 else sem_sl1)

    fire(0, 0)

    def slab_body(s, carry):
        pb = jnp.bitwise_and(s, 1)

        @pl.when(jnp.logical_and(s + 1 < nslab, pb == 0))
        def _p1():
            fire(s + 1, 1)

        @pl.when(jnp.logical_and(s + 1 < nslab, pb == 1))
        def _p0():
            fire(s + 1, 0)

        @pl.when(pb == 0)
        def _w0():
            pltpu.make_async_copy(
                tabt_hbm.at[:, pl.ds(0, SLABW)], slab_v.at[0],
                sem_sl0).wait()

        @pl.when(pb == 1)
        def _w1():
            pltpu.make_async_copy(
                tabt_hbm.at[:, pl.ds(0, SLABW)], slab_v.at[1],
                sem_sl1).wait()

        c_eff = jnp.minimum(c_lo + s * SLABC, NTC - SLABC)
        reloff = (c_eff - c_lo) * 128
        slab_ref = slab_v.at[pb]

        def scan(j, carry2):
            hv = hits_v[pl.ds(j * L, L)]
            urel16 = jnp.bitwise_and(hv, 32767)
            is_ph = hv >= (1 << 30)
            m0 = jnp.logical_and(
                jnp.logical_or(jnp.right_shift(urel16, 9) == s,
                               jnp.logical_and(is_ph, s == 0)),
                hv >= 0)

            pc = plsc.all_reduce_population_count(m0)[0]

            @pl.when(pc > 0)
            def _():
                def hit_loop(h, m):
                    lane = plsc.all_reduce_ffs(m)[0]
                    pack = hv.at[jnp.broadcast_to(lane, (L,))].get(
                        mode="promise_in_bounds")[0]
                    phantom = pack >= (1 << 30)
                    b = jnp.where(phantom,
                                  SENT + jnp.bitwise_and(pack, CHROWS - 1),
                                  jnp.right_shift(pack, 15))
                    ul = jnp.where(phantom, 0,
                                   jnp.bitwise_and(pack, 32767) - reloff)
                    rs = st_s[0]
                    slot = jnp.bitwise_and(rs, 2 * CHROWS - 1)
                    p = jnp.right_shift(jnp.bitwise_and(rs, 2 * CHROWS - 1),
                                        7)
                    @pl.when(jnp.logical_and(
                        jnp.bitwise_and(rs, CHROWS - 1) == 0,
                        rs >= 2 * CHROWS))
                    def _d():
                        pltpu.make_async_copy(
                            rows_hbm.at[pl.ds(0, CHROWS)],
                            stage_v.at[pl.ds(p * CHROWS, CHROWS)],
                            sem_scat).wait()
                        st_s[2] = st_s[2] + 1

                    ulv = jnp.broadcast_to(ul, (L,))
                    for q in range(F // L):
                        colq = plsc.load_gather(slab_ref, [fvecs[q], ulv])
                        stage_v[slot, pl.ds(q * L, L)] = colq
                    wnd = jnp.left_shift(
                        jnp.right_shift(jnp.bitwise_and(rs, CHROWS - 1), 4),
                        4)
                    tw = tags_v[p, pl.ds(wnd, L)]
                    tags_v[p, pl.ds(wnd, L)] = jnp.where(
                        lanes == jnp.bitwise_and(rs, L - 1), b, tw)
                    st_s[0] = rs + 1

                    @pl.when(jnp.bitwise_and(rs, CHROWS - 1) == CHROWS - 1)
                    def _f():
                        pltpu.async_copy(
                            stage_v.at[pl.ds(p * CHROWS, CHROWS)],
                            rows_hbm.at[tags_v.at[p]],
                            sem_scat)
                        st_s[1] = st_s[1] + 1

                    return jnp.logical_and(m, lanes != lane)

                lax.fori_loop(0, pc, hit_loop, m0)

            return carry2

        lax.fori_loop(0, nvec, scan, 0)
        return carry

    lax.fori_loop(0, nslab, slab_body, 0)

    def drain(i, carry):
        pltpu.make_async_copy(
            rows_hbm.at[pl.ds(0, CHROWS)],
            stage_v.at[pl.ds(0, CHROWS)],
            sem_scat).wait()
        return carry

    lax.fori_loop(0, st_s[1] - st_s[2], drain, 0)


@functools.partial(
    pl.kernel,
    mesh=_MESH,
    out_type=jax.ShapeDtypeStruct((B,), jnp.float32),
    compiler_params=_PARAMS,
    scratch_types=[
        pltpu.VMEM((CHROWS, 2 * F), jnp.float32),
        pltpu.VMEM((CHROWS, 2 * F), jnp.float32),
        pltpu.VMEM((BPW,), jnp.float32),
        pltpu.VMEM((L * L,), jnp.float32),
    ],
)
def _dot_rows(urows_hbm, irows_hbm, out_hbm,
              uchunk_v, ichunk_v, out_v, stage_v):
    wid = lax.axis_index("s") * NC + lax.axis_index("c")
    base = wid * BPW
    lane = lax.iota(jnp.int32, L)
    colbase = lane * L

    def chunk_body(j, carry):
        c0 = j * CHROWS
        pltpu.sync_copy(urows_hbm.at[pl.ds(base + c0, CHROWS)], uchunk_v)
        pltpu.sync_copy(irows_hbm.at[pl.ds(base + c0, CHROWS)], ichunk_v)

        def group_body(g, carry2):
            for r in range(L):
                row = g * L + r
                s = jnp.zeros((L,), jnp.float32)
                for q in range(F // L):
                    u = uchunk_v[row, pl.ds(q * L, L)]
                    v = ichunk_v[row, pl.ds(q * L, L)]
                    s = s + u * v
                stage_v[pl.ds(r * L, L)] = s
            acc = jnp.zeros((L,), jnp.float32)
            for k in range(L):
                acc = acc + plsc.load_gather(stage_v, [colbase + k])
            out_v[pl.ds(c0 + g * L, L)] = acc
            return carry2

        lax.fori_loop(0, CHROWS // L, group_body, 0)
        return carry

    lax.fori_loop(0, BPW // CHROWS, chunk_body, 0)

    pltpu.sync_copy(out_v, out_hbm.at[pl.ds(base, BPW)])


def kernel(users, items, user_embeddings, item_embeddings):
    users = users.astype(jnp.int32)
    items = items.astype(jnp.int32)
    urows = _gather_cols(users, user_embeddings.T)
    irows = _gather_cols(items, item_embeddings.T)
    return _dot_rows(urows, irows)
